# Initial kernel scaffold; baseline (speedup 1.0000x reference)
import functools

import jax
import jax.numpy as jnp
from jax import lax
from jax.experimental import pallas as pl
from jax.experimental.pallas import tpu as pltpu
from jax.experimental.pallas import tpu_sc as plsc

_N = 10000
_E = 320000
_D = 128
_K = 128
_NC = 2
_NS = 16
_NW = _NC * _NS
_CHUNKS = _E // _K
_ITERS = -(-_CHUNKS // _NW)
_RPT = _N // _NS
_BLK = 1000
_GRID = _N // _BLK


def _sc_body(with_deg, h_hbm, src_hbm, dst_hbm, zeros_hbm, zeros16_hbm,
             agg_out, deg_out, src_v, dst_v, rows_v, ones_v, agg_sh, deg_sh,
             sem):
    cid = lax.axis_index("c")
    sid = lax.axis_index("s")
    wid = sid * _NC + cid
    r0 = sid * _RPT

    pltpu.sync_copy(zeros_hbm.at[pl.ds(r0, _RPT)], agg_sh.at[pl.ds(r0, _RPT)])
    if with_deg:
        pltpu.sync_copy(zeros16_hbm.at[pl.ds(r0, _RPT)],
                        deg_sh.at[pl.ds(r0, _RPT)])

        def _ones(i, carry):
            ones_v[i, :] = jnp.ones((16,), jnp.float32)
            return carry
        lax.fori_loop(0, _K, _ones, 0)
    plsc.subcore_barrier()

    def _step(j, carry):
        c = wid + j * _NW

        @pl.when(c < _CHUNKS)
        def _():
            base = c * _K
            pltpu.sync_copy(src_hbm.at[pl.ds(base, _K)], src_v)
            pltpu.sync_copy(dst_hbm.at[pl.ds(base, _K)], dst_v)
            pltpu.async_copy(h_hbm.at[src_v], rows_v, sem).wait()
            pltpu.sync_copy(rows_v, agg_sh.at[dst_v], add=True)
            if with_deg:
                pltpu.sync_copy(ones_v, deg_sh.at[dst_v], add=True)
        return carry

    lax.fori_loop(0, _ITERS, _step, 0)
    plsc.subcore_barrier()

    pltpu.sync_copy(agg_sh.at[pl.ds(r0, _RPT)],
                    agg_out.at[cid, pl.ds(r0, _RPT)])
    if with_deg:
        pltpu.sync_copy(deg_sh.at[pl.ds(r0, _RPT)],
                        deg_out.at[cid, pl.ds(r0, _RPT)])


def _make_sc_agg(with_deg):
    mesh = plsc.VectorSubcoreMesh(core_axis_name="c", subcore_axis_name="s")
    out_type = [jax.ShapeDtypeStruct((_NC, _N, _D), jnp.float32)]
    if with_deg:
        out_type.append(jax.ShapeDtypeStruct((_NC, _N, 16), jnp.float32))
    scratch = [
        pltpu.VMEM((_K,), jnp.int32),
        pltpu.VMEM((_K,), jnp.int32),
        pltpu.VMEM((_K, _D), jnp.float32),
        pltpu.VMEM((_K, 16), jnp.float32),
        pltpu.VMEM_SHARED((_N, _D), jnp.float32),
        pltpu.VMEM_SHARED((_N, 16), jnp.float32),
        pltpu.SemaphoreType.DMA,
    ]

    if with_deg:
        def body(h, src, dst, z, z16, agg, deg, *rest):
            _sc_body(True, h, src, dst, z, z16, agg, deg, *rest)
    else:
        def body(h, src, dst, z, z16, agg, *rest):
            _sc_body(False, h, src, dst, z, z16, agg, None, *rest)

    return pl.kernel(body, out_type=out_type, mesh=mesh,
                     scratch_types=scratch)


_sc_agg_deg = _make_sc_agg(True)
_sc_agg = _make_sc_agg(False)


def _mm0_body(x_ref, w_ref, b_ref, h_ref):
    h_ref[...] = jnp.dot(x_ref[...], w_ref[...],
                         preferred_element_type=jnp.float32) + b_ref[...]


_mm0 = pl.pallas_call(
    _mm0_body,
    grid=(_GRID,),
    in_specs=[
        pl.BlockSpec((_BLK, _D), lambda i: (i, 0)),
        pl.BlockSpec((_D, _D), lambda i: (0, 0)),
        pl.BlockSpec((1, _D), lambda i: (0, 0)),
    ],
    out_specs=pl.BlockSpec((_BLK, _D), lambda i: (i, 0)),
    out_shape=jax.ShapeDtypeStruct((_N, _D), jnp.float32),
)


def _layer_body(agg_ref, deg_ref, cur_ref, w_ref, b_ref, curn_ref, h_ref):
    deg = deg_ref[0, :, 0] + deg_ref[1, :, 0]
    inv = 1.0 / jnp.maximum(deg, 1.0)
    a = agg_ref[0] + agg_ref[1]
    curn = jnp.maximum(a * inv[:, None], 0.0) + cur_ref[...]
    curn_ref[...] = curn
    h_ref[...] = jnp.dot(curn, w_ref[...],
                         preferred_element_type=jnp.float32) + b_ref[...]


_layer = pl.pallas_call(
    _layer_body,
    grid=(_GRID,),
    in_specs=[
        pl.BlockSpec((_NC, _BLK, _D), lambda i: (0, i, 0)),
        pl.BlockSpec((_NC, _BLK, 16), lambda i: (0, i, 0)),
        pl.BlockSpec((_BLK, _D), lambda i: (i, 0)),
        pl.BlockSpec((_D, _D), lambda i: (0, 0)),
        pl.BlockSpec((1, _D), lambda i: (0, 0)),
    ],
    out_specs=[
        pl.BlockSpec((_BLK, _D), lambda i: (i, 0)),
        pl.BlockSpec((_BLK, _D), lambda i: (i, 0)),
    ],
    out_shape=[
        jax.ShapeDtypeStruct((_N, _D), jnp.float32),
        jax.ShapeDtypeStruct((_N, _D), jnp.float32),
    ],
)


def _final_body(agg_ref, deg_ref, cur_ref, out_ref):
    deg = deg_ref[0, :, 0] + deg_ref[1, :, 0]
    inv = 1.0 / jnp.maximum(deg, 1.0)
    a = agg_ref[0] + agg_ref[1]
    curn = jnp.maximum(a * inv[:, None], 0.0) + cur_ref[...]
    nrm = jnp.sqrt(jnp.sum(curn * curn, axis=-1, keepdims=True))
    out_ref[...] = curn / jnp.maximum(nrm, 1e-12)


_final = pl.pallas_call(
    _final_body,
    grid=(_GRID,),
    in_specs=[
        pl.BlockSpec((_NC, _BLK, _D), lambda i: (0, i, 0)),
        pl.BlockSpec((_NC, _BLK, 16), lambda i: (0, i, 0)),
        pl.BlockSpec((_BLK, _D), lambda i: (i, 0)),
    ],
    out_specs=pl.BlockSpec((_BLK, _D), lambda i: (i, 0)),
    out_shape=jax.ShapeDtypeStruct((_N, _D), jnp.float32),
)


def kernel(x, edge_index, W0, b0, W1, b1, W2, b2):
    src = edge_index[0]
    dst = edge_index[1]
    zeros = jnp.zeros((_N, _D), jnp.float32)
    zeros16 = jnp.zeros((_N, 16), jnp.float32)

    h0 = _mm0(x, W0, b0.reshape(1, _D))
    agg0, deg = _sc_agg_deg(h0, src, dst, zeros, zeros16)
    cur1, h1 = _layer(agg0, deg, x, W1, b1.reshape(1, _D))
    (agg1,) = _sc_agg(h1, src, dst, zeros, zeros16)
    cur2, h2 = _layer(agg1, deg, cur1, W2, b2.reshape(1, _D))
    (agg2,) = _sc_agg(h2, src, dst, zeros, zeros16)
    out = _final(agg2, deg, cur2)
    return (out, edge_index)


# speedup vs baseline: 5.7227x; 5.7227x over previous
import jax
import jax.numpy as jnp
from jax import lax
from jax.experimental import pallas as pl
from jax.experimental.pallas import tpu as pltpu
from jax.experimental.pallas import tpu_sc as plsc

_N = 10000
_E = 320000
_D = 128
_K = 128
_NC = 2
_NS = 16
_NW = _NC * _NS
_CHUNKS = _E // _K
_ITERS = -(-_CHUNKS // _NW)
_NPAD = 10240
_RPT = _NPAD // _NS
_BLK = 1000
_GRID = _N // _BLK

_MESH = plsc.VectorSubcoreMesh(core_axis_name="c", subcore_axis_name="s",
                               num_cores=_NC, num_subcores=_NS)


def _agg_body(h_hbm, src_hbm, dst_hbm, zeros_hbm, agg_out,
              src_v, dst_v, rows_v, agg_sh, sem):
    cid = lax.axis_index("c")
    sid = lax.axis_index("s")
    wid = sid * _NC + cid
    r0 = sid * _RPT

    for j in range(_RPT // _K):
        pltpu.sync_copy(zeros_hbm.at[pl.ds(r0 + j * _K, _K)], rows_v)
        pltpu.sync_copy(rows_v, agg_sh.at[pl.ds(r0 + j * _K, _K)])
    plsc.subcore_barrier()

    def _step(j, carry):
        c = wid + j * _NW

        @pl.when(c < _CHUNKS)
        def _():
            base = c * _K
            pltpu.sync_copy(src_hbm.at[pl.ds(base, _K)], src_v)
            pltpu.sync_copy(dst_hbm.at[pl.ds(base, _K)], dst_v)
            pltpu.async_copy(h_hbm.at[src_v], rows_v, sem).wait()
            pltpu.sync_copy(rows_v, agg_sh.at[dst_v], add=True)
        return carry

    lax.fori_loop(0, _ITERS, _step, 0)
    plsc.subcore_barrier()

    o0 = cid * _NPAD + r0
    for j in range(_RPT // _K):
        pltpu.sync_copy(agg_sh.at[pl.ds(r0 + j * _K, _K)], rows_v)
        pltpu.sync_copy(rows_v, agg_out.at[pl.ds(o0 + j * _K, _K)])


_sc_agg = pl.kernel(
    _agg_body,
    out_type=jax.ShapeDtypeStruct((_NC * _NPAD, _D), jnp.float32),
    mesh=_MESH,
    scratch_types=[
        pltpu.VMEM((_K,), jnp.int32),
        pltpu.VMEM((_K,), jnp.int32),
        pltpu.VMEM((_K, _D), jnp.float32),
        pltpu.VMEM_SHARED((_NPAD, _D), jnp.float32),
        pltpu.SemaphoreType.DMA,
    ],
)


def _deg_body(dst_hbm, zeros_hbm, ones_hbm, deg_out,
              dst_v, ones_v, deg_sh, sem):
    cid = lax.axis_index("c")
    sid = lax.axis_index("s")
    wid = sid * _NC + cid
    r0 = sid * _RPT

    for j in range(_RPT // _K):
        pltpu.sync_copy(zeros_hbm.at[pl.ds(r0 + j * _K, _K)], ones_v)
        pltpu.sync_copy(ones_v, deg_sh.at[pl.ds(r0 + j * _K, _K)])
    pltpu.sync_copy(ones_hbm, ones_v)
    plsc.subcore_barrier()

    def _step(j, carry):
        c = wid + j * _NW

        @pl.when(c < _CHUNKS)
        def _():
            pltpu.sync_copy(dst_hbm.at[pl.ds(c * _K, _K)], dst_v)
            pltpu.sync_copy(ones_v, deg_sh.at[dst_v], add=True)
        return carry

    lax.fori_loop(0, _ITERS, _step, 0)
    plsc.subcore_barrier()

    o0 = cid * _NPAD + r0
    for j in range(_RPT // _K):
        pltpu.sync_copy(deg_sh.at[pl.ds(r0 + j * _K, _K)], ones_v)
        pltpu.sync_copy(ones_v, deg_out.at[pl.ds(o0 + j * _K, _K)])


_sc_deg = pl.kernel(
    _deg_body,
    out_type=jax.ShapeDtypeStruct((_NC * _NPAD, _D), jnp.float32),
    mesh=_MESH,
    scratch_types=[
        pltpu.VMEM((_K,), jnp.int32),
        pltpu.VMEM((_K, _D), jnp.float32),
        pltpu.VMEM_SHARED((_NPAD, _D), jnp.float32),
        pltpu.SemaphoreType.DMA,
    ],
)


def _degred_body(deg_ref, inv_ref):
    deg = deg_ref[0, :, 0] + deg_ref[1, :, 0]
    inv = 1.0 / jnp.maximum(deg, 1.0)
    inv_ref[...] = jnp.broadcast_to(inv[:, None], inv_ref.shape)


_degred = pl.pallas_call(
    _degred_body,
    grid=(_NPAD // _BLK,),
    in_specs=[pl.BlockSpec((_NC, _BLK, _D), lambda i: (0, i, 0))],
    out_specs=pl.BlockSpec((_BLK, 16), lambda i: (i, 0)),
    out_shape=jax.ShapeDtypeStruct((_NPAD, 16), jnp.float32),
)


def _mm0_body(x_ref, w_ref, b_ref, h_ref):
    h_ref[...] = jnp.dot(x_ref[...], w_ref[...],
                         preferred_element_type=jnp.float32) + b_ref[...]


_mm0 = pl.pallas_call(
    _mm0_body,
    grid=(_GRID,),
    in_specs=[
        pl.BlockSpec((_BLK, _D), lambda i: (i, 0)),
        pl.BlockSpec((_D, _D), lambda i: (0, 0)),
        pl.BlockSpec((1, _D), lambda i: (0, 0)),
    ],
    out_specs=pl.BlockSpec((_BLK, _D), lambda i: (i, 0)),
    out_shape=jax.ShapeDtypeStruct((_N, _D), jnp.float32),
)


def _layer_body(agg_ref, inv_ref, cur_ref, w_ref, b_ref, curn_ref, h_ref):
    inv = inv_ref[:, 0]
    a = agg_ref[0] + agg_ref[1]
    curn = jnp.maximum(a * inv[:, None], 0.0) + cur_ref[...]
    curn_ref[...] = curn
    h_ref[...] = jnp.dot(curn, w_ref[...],
                         preferred_element_type=jnp.float32) + b_ref[...]


_layer = pl.pallas_call(
    _layer_body,
    grid=(_GRID,),
    in_specs=[
        pl.BlockSpec((_NC, _BLK, _D), lambda i: (0, i, 0)),
        pl.BlockSpec((_BLK, 16), lambda i: (i, 0)),
        pl.BlockSpec((_BLK, _D), lambda i: (i, 0)),
        pl.BlockSpec((_D, _D), lambda i: (0, 0)),
        pl.BlockSpec((1, _D), lambda i: (0, 0)),
    ],
    out_specs=[
        pl.BlockSpec((_BLK, _D), lambda i: (i, 0)),
        pl.BlockSpec((_BLK, _D), lambda i: (i, 0)),
    ],
    out_shape=[
        jax.ShapeDtypeStruct((_N, _D), jnp.float32),
        jax.ShapeDtypeStruct((_N, _D), jnp.float32),
    ],
)


def _final_body(agg_ref, inv_ref, cur_ref, out_ref):
    inv = inv_ref[:, 0]
    a = agg_ref[0] + agg_ref[1]
    curn = jnp.maximum(a * inv[:, None], 0.0) + cur_ref[...]
    nrm = jnp.sqrt(jnp.sum(curn * curn, axis=-1, keepdims=True))
    out_ref[...] = curn / jnp.maximum(nrm, 1e-12)


_final = pl.pallas_call(
    _final_body,
    grid=(_GRID,),
    in_specs=[
        pl.BlockSpec((_NC, _BLK, _D), lambda i: (0, i, 0)),
        pl.BlockSpec((_BLK, 16), lambda i: (i, 0)),
        pl.BlockSpec((_BLK, _D), lambda i: (i, 0)),
    ],
    out_specs=pl.BlockSpec((_BLK, _D), lambda i: (i, 0)),
    out_shape=jax.ShapeDtypeStruct((_N, _D), jnp.float32),
)


def kernel(x, edge_index, W0, b0, W1, b1, W2, b2):
    src = edge_index[0]
    dst = edge_index[1]
    zeros = jnp.zeros((_NPAD, _D), jnp.float32)
    ones = jnp.ones((_K, _D), jnp.float32)

    deg_fat = _sc_deg(dst, zeros, ones).reshape(_NC, _NPAD, _D)
    inv = _degred(deg_fat)
    h0 = _mm0(x, W0, b0.reshape(1, _D))
    agg0 = _sc_agg(h0, src, dst, zeros).reshape(_NC, _NPAD, _D)
    cur1, h1 = _layer(agg0, inv, x, W1, b1.reshape(1, _D))
    agg1 = _sc_agg(h1, src, dst, zeros).reshape(_NC, _NPAD, _D)
    cur2, h2 = _layer(agg1, inv, cur1, W2, b2.reshape(1, _D))
    agg2 = _sc_agg(h2, src, dst, zeros).reshape(_NC, _NPAD, _D)
    out = _final(agg2, inv, cur2)
    return (out, edge_index)


# speedup vs baseline: 8.0459x; 1.4060x over previous
import jax
import jax.numpy as jnp
from jax import lax
from jax.experimental import pallas as pl
from jax.experimental.pallas import tpu as pltpu
from jax.experimental.pallas import tpu_sc as plsc

_N = 10000
_E = 320000
_D = 128
_K = 128
_NC = 2
_NS = 16
_NW = _NC * _NS
_CHUNKS = _E // _K
_ITERS = -(-_CHUNKS // _NW)
_KA = 64
_CHUNKS_A = _E // _KA
_ITERS_A = -(-_CHUNKS_A // _NW)
_ROUNDS_A = -(-_ITERS_A // 2)
_NPAD = 10240
_RPT = _NPAD // _NS
_BLK = 1000
_GRID = _N // _BLK

_MESH = plsc.VectorSubcoreMesh(core_axis_name="c", subcore_axis_name="s",
                               num_cores=_NC, num_subcores=_NS)


_G = 8
_GROUPS = _CHUNKS_A // _G
_GPW = -(-_GROUPS // _NW)
_PAIRS = -(-_GPW // 2)


def _agg_body(h_hbm, src_hbm, dst_hbm, zeros_hbm, agg_out,
              srcA, dstA, srcB, dstB, rows0, rows1, agg_sh,
              gsem0, gsem1):
    cid = lax.axis_index("c")
    sid = lax.axis_index("s")
    wid = sid * _NC + cid
    r0 = sid * _RPT

    for j in range(_RPT // _KA):
        pltpu.sync_copy(zeros_hbm.at[pl.ds(r0 + j * _KA, _KA)], rows0)
        pltpu.sync_copy(rows0, agg_sh.at[pl.ds(r0 + j * _KA, _KA)])
    plsc.subcore_barrier()

    rows = (rows0, rows1)
    sems = (gsem0, gsem1)

    def _load_idx(g, src_g, dst_g):
        pltpu.sync_copy(src_hbm.at[pl.ds(g * _G, _G)], src_g)
        pltpu.sync_copy(dst_hbm.at[pl.ds(g * _G, _G)], dst_g)

    _load_idx(wid, srcA, dstA)
    pltpu.async_copy(h_hbm.at[srcA.at[0]], rows[0], gsem0)

    def _pair(R, carry):
        gA = wid + (2 * R) * _NW
        gB = gA + _NW
        gA_next = gA + 2 * _NW

        @pl.when(gB < _GROUPS)
        def _():
            _load_idx(gB, srcB, dstB)

        for t in range(2 * _G):
            if t == _G:
                @pl.when(gA_next < _GROUPS)
                def _():
                    _load_idx(gA_next, srcA, dstA)

            tn = t + 1
            if tn < _G:
                nsrc, npred_row = srcA, tn
            elif tn < 2 * _G:
                nsrc, npred_row = srcB, tn - _G
            else:
                nsrc, npred_row = srcA, 0
            npred = (gA < _GROUPS) if tn < _G else (
                (gB < _GROUPS) if tn < 2 * _G else (gA_next < _GROUPS))

            @pl.when(npred)
            def _():
                pltpu.async_copy(h_hbm.at[nsrc.at[npred_row]],
                                 rows[tn % 2], sems[tn % 2])

            cpred = (gA < _GROUPS) if t < _G else (gB < _GROUPS)
            cdst = dstA.at[t] if t < _G else dstB.at[t - _G]
            csrc = srcA.at[t] if t < _G else srcB.at[t - _G]

            @pl.when(cpred)
            def _():
                pltpu.make_async_copy(h_hbm.at[csrc], rows[t % 2],
                                      sems[t % 2]).wait()
                pltpu.sync_copy(rows[t % 2], agg_sh.at[cdst], add=True)
        return carry

    lax.fori_loop(0, _PAIRS, _pair, 0)
    plsc.subcore_barrier()

    o0 = cid * _NPAD + r0
    for j in range(_RPT // _KA):
        pltpu.sync_copy(agg_sh.at[pl.ds(r0 + j * _KA, _KA)], rows0)
        pltpu.sync_copy(rows0, agg_out.at[pl.ds(o0 + j * _KA, _KA)])


_sc_agg = pl.kernel(
    _agg_body,
    out_type=jax.ShapeDtypeStruct((_NC * _NPAD, _D), jnp.float32),
    mesh=_MESH,
    scratch_types=[
        pltpu.VMEM((_G, _KA), jnp.int32),
        pltpu.VMEM((_G, _KA), jnp.int32),
        pltpu.VMEM((_G, _KA), jnp.int32),
        pltpu.VMEM((_G, _KA), jnp.int32),
        pltpu.VMEM((_KA, _D), jnp.float32),
        pltpu.VMEM((_KA, _D), jnp.float32),
        pltpu.VMEM_SHARED((_NPAD, _D), jnp.float32),
        pltpu.SemaphoreType.DMA,
        pltpu.SemaphoreType.DMA,
    ],
)


def _deg_body(dst_hbm, zeros_hbm, ones_hbm, deg_out,
              dst_v, ones_v, deg_sh, sem):
    cid = lax.axis_index("c")
    sid = lax.axis_index("s")
    wid = sid * _NC + cid
    r0 = sid * _RPT

    for j in range(_RPT // _K):
        pltpu.sync_copy(zeros_hbm.at[pl.ds(r0 + j * _K, _K)], ones_v)
        pltpu.sync_copy(ones_v, deg_sh.at[pl.ds(r0 + j * _K, _K)])
    pltpu.sync_copy(ones_hbm, ones_v)
    plsc.subcore_barrier()

    def _step(j, carry):
        c = wid + j * _NW

        @pl.when(c < _CHUNKS)
        def _():
            pltpu.sync_copy(dst_hbm.at[pl.ds(c * _K, _K)], dst_v)
            pltpu.sync_copy(ones_v, deg_sh.at[dst_v], add=True)
        return carry

    lax.fori_loop(0, _ITERS, _step, 0)
    plsc.subcore_barrier()

    o0 = cid * _NPAD + r0
    for j in range(_RPT // _K):
        pltpu.sync_copy(deg_sh.at[pl.ds(r0 + j * _K, _K)], ones_v)
        pltpu.sync_copy(ones_v, deg_out.at[pl.ds(o0 + j * _K, _K)])


_sc_deg = pl.kernel(
    _deg_body,
    out_type=jax.ShapeDtypeStruct((_NC * _NPAD, _D), jnp.float32),
    mesh=_MESH,
    scratch_types=[
        pltpu.VMEM((_K,), jnp.int32),
        pltpu.VMEM((_K, _D), jnp.float32),
        pltpu.VMEM_SHARED((_NPAD, _D), jnp.float32),
        pltpu.SemaphoreType.DMA,
    ],
)


def _degred_body(deg_ref, inv_ref):
    deg = deg_ref[0, :, 0] + deg_ref[1, :, 0]
    inv = 1.0 / jnp.maximum(deg, 1.0)
    inv_ref[...] = jnp.broadcast_to(inv[:, None], inv_ref.shape)


_degred = pl.pallas_call(
    _degred_body,
    grid=(_NPAD // _BLK,),
    in_specs=[pl.BlockSpec((_NC, _BLK, _D), lambda i: (0, i, 0))],
    out_specs=pl.BlockSpec((_BLK, 16), lambda i: (i, 0)),
    out_shape=jax.ShapeDtypeStruct((_NPAD, 16), jnp.float32),
)


def _mm0_body(x_ref, w_ref, b_ref, h_ref):
    h_ref[...] = jnp.dot(x_ref[...], w_ref[...],
                         preferred_element_type=jnp.float32) + b_ref[...]


_mm0 = pl.pallas_call(
    _mm0_body,
    grid=(_GRID,),
    in_specs=[
        pl.BlockSpec((_BLK, _D), lambda i: (i, 0)),
        pl.BlockSpec((_D, _D), lambda i: (0, 0)),
        pl.BlockSpec((1, _D), lambda i: (0, 0)),
    ],
    out_specs=pl.BlockSpec((_BLK, _D), lambda i: (i, 0)),
    out_shape=jax.ShapeDtypeStruct((_N, _D), jnp.float32),
)


def _layer_body(agg_ref, inv_ref, cur_ref, w_ref, b_ref, curn_ref, h_ref):
    inv = inv_ref[:, 0]
    a = agg_ref[0] + agg_ref[1]
    curn = jnp.maximum(a * inv[:, None], 0.0) + cur_ref[...]
    curn_ref[...] = curn
    h_ref[...] = jnp.dot(curn, w_ref[...],
                         preferred_element_type=jnp.float32) + b_ref[...]


_layer = pl.pallas_call(
    _layer_body,
    grid=(_GRID,),
    in_specs=[
        pl.BlockSpec((_NC, _BLK, _D), lambda i: (0, i, 0)),
        pl.BlockSpec((_BLK, 16), lambda i: (i, 0)),
        pl.BlockSpec((_BLK, _D), lambda i: (i, 0)),
        pl.BlockSpec((_D, _D), lambda i: (0, 0)),
        pl.BlockSpec((1, _D), lambda i: (0, 0)),
    ],
    out_specs=[
        pl.BlockSpec((_BLK, _D), lambda i: (i, 0)),
        pl.BlockSpec((_BLK, _D), lambda i: (i, 0)),
    ],
    out_shape=[
        jax.ShapeDtypeStruct((_N, _D), jnp.float32),
        jax.ShapeDtypeStruct((_N, _D), jnp.float32),
    ],
)


def _final_body(agg_ref, inv_ref, cur_ref, out_ref):
    inv = inv_ref[:, 0]
    a = agg_ref[0] + agg_ref[1]
    curn = jnp.maximum(a * inv[:, None], 0.0) + cur_ref[...]
    nrm = jnp.sqrt(jnp.sum(curn * curn, axis=-1, keepdims=True))
    out_ref[...] = curn / jnp.maximum(nrm, 1e-12)


_final = pl.pallas_call(
    _final_body,
    grid=(_GRID,),
    in_specs=[
        pl.BlockSpec((_NC, _BLK, _D), lambda i: (0, i, 0)),
        pl.BlockSpec((_BLK, 16), lambda i: (i, 0)),
        pl.BlockSpec((_BLK, _D), lambda i: (i, 0)),
    ],
    out_specs=pl.BlockSpec((_BLK, _D), lambda i: (i, 0)),
    out_shape=jax.ShapeDtypeStruct((_N, _D), jnp.float32),
)


def kernel(x, edge_index, W0, b0, W1, b1, W2, b2):
    src = edge_index[0]
    dst = edge_index[1]
    zeros = jnp.zeros((_NPAD, _D), jnp.float32)
    ones = jnp.ones((_K, _D), jnp.float32)

    src2d = src.reshape(_CHUNKS_A, _KA)
    dst2d = dst.reshape(_CHUNKS_A, _KA)

    deg_fat = _sc_deg(dst, zeros, ones).reshape(_NC, _NPAD, _D)
    inv = _degred(deg_fat)
    h0 = _mm0(x, W0, b0.reshape(1, _D))
    agg0 = _sc_agg(h0, src2d, dst2d, zeros).reshape(_NC, _NPAD, _D)
    cur1, h1 = _layer(agg0, inv, x, W1, b1.reshape(1, _D))
    agg1 = _sc_agg(h1, src2d, dst2d, zeros).reshape(_NC, _NPAD, _D)
    cur2, h2 = _layer(agg1, inv, cur1, W2, b2.reshape(1, _D))
    agg2 = _sc_agg(h2, src2d, dst2d, zeros).reshape(_NC, _NPAD, _D)
    out = _final(agg2, inv, cur2)
    return (out, edge_index)


# speedup vs baseline: 8.7041x; 1.0818x over previous
import jax
import jax.numpy as jnp
from jax import lax
from jax.experimental import pallas as pl
from jax.experimental.pallas import tpu as pltpu
from jax.experimental.pallas import tpu_sc as plsc

_N = 10000
_E = 320000
_D = 128
_K = 128
_NC = 2
_NS = 16
_NW = _NC * _NS
_CHUNKS = _E // _K
_ITERS = -(-_CHUNKS // _NW)
_KA = 80
_CHUNKS_A = _E // _KA
_ITERS_A = -(-_CHUNKS_A // _NW)
_ROUNDS_A = -(-_ITERS_A // 2)
_NPAD = 10240
_RPT = _NPAD // _NS
_BLK = 1000
_GRID = _N // _BLK

_MESH = plsc.VectorSubcoreMesh(core_axis_name="c", subcore_axis_name="s",
                               num_cores=_NC, num_subcores=_NS)


_G = 8
_GROUPS = _CHUNKS_A // _G
_GPW = -(-_GROUPS // _NW)
_PAIRS = -(-_GPW // 2)


def _agg_body(h_hbm, src_hbm, dst_hbm, zeros_hbm, agg_out,
              srcA, dstA, srcB, dstB, rows0, rows1, agg_sh,
              gsem0, gsem1):
    cid = lax.axis_index("c")
    sid = lax.axis_index("s")
    wid = sid * _NC + cid
    r0 = sid * _RPT

    for j in range(_RPT // _KA):
        pltpu.sync_copy(zeros_hbm.at[pl.ds(r0 + j * _KA, _KA)], rows0)
        pltpu.sync_copy(rows0, agg_sh.at[pl.ds(r0 + j * _KA, _KA)])
    plsc.subcore_barrier()

    rows = (rows0, rows1)
    sems = (gsem0, gsem1)

    def _load_idx(g, src_g, dst_g):
        pltpu.sync_copy(src_hbm.at[pl.ds(g * _G, _G)], src_g)
        pltpu.sync_copy(dst_hbm.at[pl.ds(g * _G, _G)], dst_g)

    _load_idx(wid, srcA, dstA)
    pltpu.async_copy(h_hbm.at[srcA.at[0]], rows[0], gsem0)

    def _pair(R, carry):
        gA = wid + (2 * R) * _NW
        gB = gA + _NW
        gA_next = gA + 2 * _NW

        @pl.when(gB < _GROUPS)
        def _():
            _load_idx(gB, srcB, dstB)

        for t in range(2 * _G):
            if t == _G:
                @pl.when(gA_next < _GROUPS)
                def _():
                    _load_idx(gA_next, srcA, dstA)

            tn = t + 1
            if tn < _G:
                nsrc, npred_row = srcA, tn
            elif tn < 2 * _G:
                nsrc, npred_row = srcB, tn - _G
            else:
                nsrc, npred_row = srcA, 0
            npred = (gA < _GROUPS) if tn < _G else (
                (gB < _GROUPS) if tn < 2 * _G else (gA_next < _GROUPS))

            @pl.when(npred)
            def _():
                pltpu.async_copy(h_hbm.at[nsrc.at[npred_row]],
                                 rows[tn % 2], sems[tn % 2])

            cpred = (gA < _GROUPS) if t < _G else (gB < _GROUPS)
            cdst = dstA.at[t] if t < _G else dstB.at[t - _G]
            csrc = srcA.at[t] if t < _G else srcB.at[t - _G]

            @pl.when(cpred)
            def _():
                pltpu.make_async_copy(h_hbm.at[csrc], rows[t % 2],
                                      sems[t % 2]).wait()
                pltpu.sync_copy(rows[t % 2], agg_sh.at[cdst], add=True)
        return carry

    lax.fori_loop(0, _PAIRS, _pair, 0)
    plsc.subcore_barrier()

    o0 = cid * _NPAD + r0
    for j in range(_RPT // _KA):
        pltpu.sync_copy(agg_sh.at[pl.ds(r0 + j * _KA, _KA)], rows0)
        pltpu.sync_copy(rows0, agg_out.at[pl.ds(o0 + j * _KA, _KA)])


_sc_agg = pl.kernel(
    _agg_body,
    out_type=jax.ShapeDtypeStruct((_NC * _NPAD, _D), jnp.float32),
    mesh=_MESH,
    scratch_types=[
        pltpu.VMEM((_G, _KA), jnp.int32),
        pltpu.VMEM((_G, _KA), jnp.int32),
        pltpu.VMEM((_G, _KA), jnp.int32),
        pltpu.VMEM((_G, _KA), jnp.int32),
        pltpu.VMEM((_KA, _D), jnp.float32),
        pltpu.VMEM((_KA, _D), jnp.float32),
        pltpu.VMEM_SHARED((_NPAD, _D), jnp.float32),
        pltpu.SemaphoreType.DMA,
        pltpu.SemaphoreType.DMA,
    ],
)


def _deg_body(dst_hbm, zeros_hbm, ones_hbm, deg_out,
              dst_v, ones_v, deg_sh, sem):
    cid = lax.axis_index("c")
    sid = lax.axis_index("s")
    wid = sid * _NC + cid
    r0 = sid * _RPT

    for j in range(_RPT // _K):
        pltpu.sync_copy(zeros_hbm.at[pl.ds(r0 + j * _K, _K)], ones_v)
        pltpu.sync_copy(ones_v, deg_sh.at[pl.ds(r0 + j * _K, _K)])
    pltpu.sync_copy(ones_hbm, ones_v)
    plsc.subcore_barrier()

    def _step(j, carry):
        c = wid + j * _NW

        @pl.when(c < _CHUNKS)
        def _():
            pltpu.sync_copy(dst_hbm.at[pl.ds(c * _K, _K)], dst_v)
            pltpu.sync_copy(ones_v, deg_sh.at[dst_v], add=True)
        return carry

    lax.fori_loop(0, _ITERS, _step, 0)
    plsc.subcore_barrier()

    o0 = cid * _NPAD + r0
    for j in range(_RPT // _K):
        pltpu.sync_copy(deg_sh.at[pl.ds(r0 + j * _K, _K)], ones_v)
        pltpu.sync_copy(ones_v, deg_out.at[pl.ds(o0 + j * _K, _K)])


_sc_deg = pl.kernel(
    _deg_body,
    out_type=jax.ShapeDtypeStruct((_NC * _NPAD, _D), jnp.float32),
    mesh=_MESH,
    scratch_types=[
        pltpu.VMEM((_K,), jnp.int32),
        pltpu.VMEM((_K, _D), jnp.float32),
        pltpu.VMEM_SHARED((_NPAD, _D), jnp.float32),
        pltpu.SemaphoreType.DMA,
    ],
)


def _degred_body(deg_ref, inv_ref):
    deg = deg_ref[0, :, 0] + deg_ref[1, :, 0]
    inv = 1.0 / jnp.maximum(deg, 1.0)
    inv_ref[...] = jnp.broadcast_to(inv[:, None], inv_ref.shape)


_degred = pl.pallas_call(
    _degred_body,
    grid=(_NPAD // _BLK,),
    in_specs=[pl.BlockSpec((_NC, _BLK, _D), lambda i: (0, i, 0))],
    out_specs=pl.BlockSpec((_BLK, 16), lambda i: (i, 0)),
    out_shape=jax.ShapeDtypeStruct((_NPAD, 16), jnp.float32),
)


def _mm0_body(x_ref, w_ref, b_ref, h_ref):
    h_ref[...] = jnp.dot(x_ref[...], w_ref[...],
                         preferred_element_type=jnp.float32) + b_ref[...]


_mm0 = pl.pallas_call(
    _mm0_body,
    grid=(_GRID,),
    in_specs=[
        pl.BlockSpec((_BLK, _D), lambda i: (i, 0)),
        pl.BlockSpec((_D, _D), lambda i: (0, 0)),
        pl.BlockSpec((1, _D), lambda i: (0, 0)),
    ],
    out_specs=pl.BlockSpec((_BLK, _D), lambda i: (i, 0)),
    out_shape=jax.ShapeDtypeStruct((_N, _D), jnp.float32),
)


def _layer_body(agg_ref, inv_ref, cur_ref, w_ref, b_ref, curn_ref, h_ref):
    inv = inv_ref[:, 0]
    a = agg_ref[0] + agg_ref[1]
    curn = jnp.maximum(a * inv[:, None], 0.0) + cur_ref[...]
    curn_ref[...] = curn
    h_ref[...] = jnp.dot(curn, w_ref[...],
                         preferred_element_type=jnp.float32) + b_ref[...]


_layer = pl.pallas_call(
    _layer_body,
    grid=(_GRID,),
    in_specs=[
        pl.BlockSpec((_NC, _BLK, _D), lambda i: (0, i, 0)),
        pl.BlockSpec((_BLK, 16), lambda i: (i, 0)),
        pl.BlockSpec((_BLK, _D), lambda i: (i, 0)),
        pl.BlockSpec((_D, _D), lambda i: (0, 0)),
        pl.BlockSpec((1, _D), lambda i: (0, 0)),
    ],
    out_specs=[
        pl.BlockSpec((_BLK, _D), lambda i: (i, 0)),
        pl.BlockSpec((_BLK, _D), lambda i: (i, 0)),
    ],
    out_shape=[
        jax.ShapeDtypeStruct((_N, _D), jnp.float32),
        jax.ShapeDtypeStruct((_N, _D), jnp.float32),
    ],
)


def _final_body(agg_ref, inv_ref, cur_ref, out_ref):
    inv = inv_ref[:, 0]
    a = agg_ref[0] + agg_ref[1]
    curn = jnp.maximum(a * inv[:, None], 0.0) + cur_ref[...]
    nrm = jnp.sqrt(jnp.sum(curn * curn, axis=-1, keepdims=True))
    out_ref[...] = curn / jnp.maximum(nrm, 1e-12)


_final = pl.pallas_call(
    _final_body,
    grid=(_GRID,),
    in_specs=[
        pl.BlockSpec((_NC, _BLK, _D), lambda i: (0, i, 0)),
        pl.BlockSpec((_BLK, 16), lambda i: (i, 0)),
        pl.BlockSpec((_BLK, _D), lambda i: (i, 0)),
    ],
    out_specs=pl.BlockSpec((_BLK, _D), lambda i: (i, 0)),
    out_shape=jax.ShapeDtypeStruct((_N, _D), jnp.float32),
)


def kernel(x, edge_index, W0, b0, W1, b1, W2, b2):
    src = edge_index[0]
    dst = edge_index[1]
    zeros = jnp.zeros((_NPAD, _D), jnp.float32)
    ones = jnp.ones((_K, _D), jnp.float32)

    src2d = src.reshape(_CHUNKS_A, _KA)
    dst2d = dst.reshape(_CHUNKS_A, _KA)

    deg_fat = _sc_deg(dst, zeros, ones).reshape(_NC, _NPAD, _D)
    inv = _degred(deg_fat)
    h0 = _mm0(x, W0, b0.reshape(1, _D))
    agg0 = _sc_agg(h0, src2d, dst2d, zeros).reshape(_NC, _NPAD, _D)
    cur1, h1 = _layer(agg0, inv, x, W1, b1.reshape(1, _D))
    agg1 = _sc_agg(h1, src2d, dst2d, zeros).reshape(_NC, _NPAD, _D)
    cur2, h2 = _layer(agg1, inv, cur1, W2, b2.reshape(1, _D))
    agg2 = _sc_agg(h2, src2d, dst2d, zeros).reshape(_NC, _NPAD, _D)
    out = _final(agg2, inv, cur2)
    return (out, edge_index)


# speedup vs baseline: 9.0274x; 1.0371x over previous
import jax
import jax.numpy as jnp
from jax import lax
from jax.experimental import pallas as pl
from jax.experimental.pallas import tpu as pltpu
from jax.experimental.pallas import tpu_sc as plsc

_N = 10000
_E = 320000
_D = 128
_K = 128
_NC = 2
_NS = 16
_NW = _NC * _NS
_CHUNKS = _E // _K
_ITERS = -(-_CHUNKS // _NW)
_KA = 80
_CHUNKS_A = _E // _KA
_ITERS_A = -(-_CHUNKS_A // _NW)
_ROUNDS_A = -(-_ITERS_A // 2)
_NPAD = 10240
_RPT = _NPAD // _NS
_BLK = 1000
_GRID = _N // _BLK

_MESH = plsc.VectorSubcoreMesh(core_axis_name="c", subcore_axis_name="s",
                               num_cores=_NC, num_subcores=_NS)


_G = 8
_GROUPS = _CHUNKS_A // _G
_GPW = -(-_GROUPS // _NW)
_PAIRS = -(-_GPW // 2)


def _agg_body(h_hbm, src_hbm, dst_hbm, zeros_hbm, agg_out,
              srcA, dstA, srcB, dstB, rows0, rows1, agg_sh,
              gsem0, gsem1):
    cid = lax.axis_index("c")
    sid = lax.axis_index("s")
    wid = sid * _NC + cid
    r0 = sid * _RPT

    for j in range(_RPT // _KA):
        pltpu.sync_copy(zeros_hbm.at[pl.ds(r0 + j * _KA, _KA)], rows0)
        pltpu.sync_copy(rows0, agg_sh.at[pl.ds(r0 + j * _KA, _KA)])
    plsc.subcore_barrier()

    rows = (rows0, rows1)
    sems = (gsem0, gsem1)

    def _load_idx(g, src_g, dst_g):
        pltpu.sync_copy(src_hbm.at[pl.ds(g * _G, _G)], src_g)
        pltpu.sync_copy(dst_hbm.at[pl.ds(g * _G, _G)], dst_g)

    _load_idx(wid, srcA, dstA)
    pltpu.async_copy(h_hbm.at[srcA.at[0]], rows[0], gsem0)

    def _pair(R, carry):
        gA = wid + (2 * R) * _NW
        gB = gA + _NW
        gA_next = gA + 2 * _NW

        @pl.when(gB < _GROUPS)
        def _():
            _load_idx(gB, srcB, dstB)

        for t in range(2 * _G):
            if t == _G:
                @pl.when(gA_next < _GROUPS)
                def _():
                    _load_idx(gA_next, srcA, dstA)

            tn = t + 1
            if tn < _G:
                nsrc, npred_row = srcA, tn
            elif tn < 2 * _G:
                nsrc, npred_row = srcB, tn - _G
            else:
                nsrc, npred_row = srcA, 0
            npred = (gA < _GROUPS) if tn < _G else (
                (gB < _GROUPS) if tn < 2 * _G else (gA_next < _GROUPS))

            @pl.when(npred)
            def _():
                pltpu.async_copy(h_hbm.at[nsrc.at[npred_row]],
                                 rows[tn % 2], sems[tn % 2])

            cpred = (gA < _GROUPS) if t < _G else (gB < _GROUPS)
            cdst = dstA.at[t] if t < _G else dstB.at[t - _G]
            csrc = srcA.at[t] if t < _G else srcB.at[t - _G]

            @pl.when(cpred)
            def _():
                pltpu.make_async_copy(h_hbm.at[csrc], rows[t % 2],
                                      sems[t % 2]).wait()
                pltpu.sync_copy(rows[t % 2], agg_sh.at[cdst], add=True)
        return carry

    lax.fori_loop(0, _PAIRS, _pair, 0)
    plsc.subcore_barrier()

    o0 = cid * _NPAD + r0
    for j in range(_RPT // _KA):
        pltpu.sync_copy(agg_sh.at[pl.ds(r0 + j * _KA, _KA)], rows0)
        pltpu.sync_copy(rows0, agg_out.at[pl.ds(o0 + j * _KA, _KA)])


_sc_agg = pl.kernel(
    _agg_body,
    out_type=jax.ShapeDtypeStruct((_NC * _NPAD, _D), jnp.float32),
    mesh=_MESH,
    scratch_types=[
        pltpu.VMEM((_G, _KA), jnp.int32),
        pltpu.VMEM((_G, _KA), jnp.int32),
        pltpu.VMEM((_G, _KA), jnp.int32),
        pltpu.VMEM((_G, _KA), jnp.int32),
        pltpu.VMEM((_KA, _D), jnp.float32),
        pltpu.VMEM((_KA, _D), jnp.float32),
        pltpu.VMEM_SHARED((_NPAD, _D), jnp.float32),
        pltpu.SemaphoreType.DMA,
        pltpu.SemaphoreType.DMA,
    ],
)


def _deg_body(dst_hbm, zeros_hbm, ones_hbm, deg_out,
              dstG, ones_v, deg_sh, sem):
    cid = lax.axis_index("c")
    sid = lax.axis_index("s")
    wid = sid * _NC + cid
    r0 = sid * _RPT

    for j in range(_RPT // _KA):
        pltpu.sync_copy(zeros_hbm.at[pl.ds(r0 + j * _KA, _KA)], ones_v)
        pltpu.sync_copy(ones_v, deg_sh.at[pl.ds(r0 + j * _KA, _KA)])
    pltpu.sync_copy(ones_hbm, ones_v)
    plsc.subcore_barrier()

    def _step(G, carry):
        g = wid + G * _NW

        @pl.when(g < _GROUPS)
        def _():
            pltpu.sync_copy(dst_hbm.at[pl.ds(g * _G, _G)], dstG)
            for t in range(_G):
                pltpu.sync_copy(ones_v, deg_sh.at[dstG.at[t]], add=True)
        return carry

    lax.fori_loop(0, _GPW, _step, 0)
    plsc.subcore_barrier()

    o0 = cid * _NPAD + r0
    for j in range(_RPT // _KA):
        pltpu.sync_copy(deg_sh.at[pl.ds(r0 + j * _KA, _KA)], ones_v)
        pltpu.sync_copy(ones_v, deg_out.at[pl.ds(o0 + j * _KA, _KA)])


_sc_deg = pl.kernel(
    _deg_body,
    out_type=jax.ShapeDtypeStruct((_NC * _NPAD, _D), jnp.float32),
    mesh=_MESH,
    scratch_types=[
        pltpu.VMEM((_G, _KA), jnp.int32),
        pltpu.VMEM((_KA, _D), jnp.float32),
        pltpu.VMEM_SHARED((_NPAD, _D), jnp.float32),
        pltpu.SemaphoreType.DMA,
    ],
)


def _degred_body(deg_ref, inv_ref):
    deg = deg_ref[0, :, 0] + deg_ref[1, :, 0]
    inv = 1.0 / jnp.maximum(deg, 1.0)
    inv_ref[...] = jnp.broadcast_to(inv[:, None], inv_ref.shape)


_degred = pl.pallas_call(
    _degred_body,
    grid=(_NPAD // _BLK,),
    in_specs=[pl.BlockSpec((_NC, _BLK, _D), lambda i: (0, i, 0))],
    out_specs=pl.BlockSpec((_BLK, 16), lambda i: (i, 0)),
    out_shape=jax.ShapeDtypeStruct((_NPAD, 16), jnp.float32),
)


def _mm0_body(x_ref, w_ref, b_ref, h_ref):
    h_ref[...] = jnp.dot(x_ref[...], w_ref[...],
                         preferred_element_type=jnp.float32) + b_ref[...]


_mm0 = pl.pallas_call(
    _mm0_body,
    grid=(_GRID,),
    in_specs=[
        pl.BlockSpec((_BLK, _D), lambda i: (i, 0)),
        pl.BlockSpec((_D, _D), lambda i: (0, 0)),
        pl.BlockSpec((1, _D), lambda i: (0, 0)),
    ],
    out_specs=pl.BlockSpec((_BLK, _D), lambda i: (i, 0)),
    out_shape=jax.ShapeDtypeStruct((_N, _D), jnp.float32),
)


def _layer_body(agg_ref, inv_ref, cur_ref, w_ref, b_ref, curn_ref, h_ref):
    inv = inv_ref[:, 0]
    a = agg_ref[0] + agg_ref[1]
    curn = jnp.maximum(a * inv[:, None], 0.0) + cur_ref[...]
    curn_ref[...] = curn
    h_ref[...] = jnp.dot(curn, w_ref[...],
                         preferred_element_type=jnp.float32) + b_ref[...]


_layer = pl.pallas_call(
    _layer_body,
    grid=(_GRID,),
    in_specs=[
        pl.BlockSpec((_NC, _BLK, _D), lambda i: (0, i, 0)),
        pl.BlockSpec((_BLK, 16), lambda i: (i, 0)),
        pl.BlockSpec((_BLK, _D), lambda i: (i, 0)),
        pl.BlockSpec((_D, _D), lambda i: (0, 0)),
        pl.BlockSpec((1, _D), lambda i: (0, 0)),
    ],
    out_specs=[
        pl.BlockSpec((_BLK, _D), lambda i: (i, 0)),
        pl.BlockSpec((_BLK, _D), lambda i: (i, 0)),
    ],
    out_shape=[
        jax.ShapeDtypeStruct((_N, _D), jnp.float32),
        jax.ShapeDtypeStruct((_N, _D), jnp.float32),
    ],
)


def _final_body(agg_ref, inv_ref, cur_ref, out_ref):
    inv = inv_ref[:, 0]
    a = agg_ref[0] + agg_ref[1]
    curn = jnp.maximum(a * inv[:, None], 0.0) + cur_ref[...]
    nrm = jnp.sqrt(jnp.sum(curn * curn, axis=-1, keepdims=True))
    out_ref[...] = curn / jnp.maximum(nrm, 1e-12)


_final = pl.pallas_call(
    _final_body,
    grid=(_GRID,),
    in_specs=[
        pl.BlockSpec((_NC, _BLK, _D), lambda i: (0, i, 0)),
        pl.BlockSpec((_BLK, 16), lambda i: (i, 0)),
        pl.BlockSpec((_BLK, _D), lambda i: (i, 0)),
    ],
    out_specs=pl.BlockSpec((_BLK, _D), lambda i: (i, 0)),
    out_shape=jax.ShapeDtypeStruct((_N, _D), jnp.float32),
)


def kernel(x, edge_index, W0, b0, W1, b1, W2, b2):
    src = edge_index[0]
    dst = edge_index[1]
    zeros = jnp.zeros((_NPAD, _D), jnp.float32)
    ones = jnp.ones((_KA, _D), jnp.float32)

    src2d = src.reshape(_CHUNKS_A, _KA)
    dst2d = dst.reshape(_CHUNKS_A, _KA)

    deg_fat = _sc_deg(dst2d, zeros, ones).reshape(_NC, _NPAD, _D)
    inv = _degred(deg_fat)
    h0 = _mm0(x, W0, b0.reshape(1, _D))
    agg0 = _sc_agg(h0, src2d, dst2d, zeros).reshape(_NC, _NPAD, _D)
    cur1, h1 = _layer(agg0, inv, x, W1, b1.reshape(1, _D))
    agg1 = _sc_agg(h1, src2d, dst2d, zeros).reshape(_NC, _NPAD, _D)
    cur2, h2 = _layer(agg1, inv, cur1, W2, b2.reshape(1, _D))
    agg2 = _sc_agg(h2, src2d, dst2d, zeros).reshape(_NC, _NPAD, _D)
    out = _final(agg2, inv, cur2)
    return (out, edge_index)


# speedup vs baseline: 9.5192x; 1.0545x over previous
import jax
import jax.numpy as jnp
from jax import lax
from jax.experimental import pallas as pl
from jax.experimental.pallas import tpu as pltpu
from jax.experimental.pallas import tpu_sc as plsc

_N = 10000
_E = 320000
_D = 128
_K = 128
_NC = 2
_NS = 16
_NW = _NC * _NS
_CHUNKS = _E // _K
_ITERS = -(-_CHUNKS // _NW)
_KA = 100
_CHUNKS_A = _E // _KA
_NPAD = 10112
_RPT = _NPAD // _NS
_STAGE = (96, 96, 96, 96, 96, 96, 56)
_BLK = 1000
_GRID = _N // _BLK

_MESH = plsc.VectorSubcoreMesh(core_axis_name="c", subcore_axis_name="s",
                               num_cores=_NC, num_subcores=_NS)


_G = 8
_GROUPS = _CHUNKS_A // _G
_GPW = -(-_GROUPS // _NW)
_PAIRS = -(-_GPW // 2)


def _agg_body(h_hbm, src_hbm, dst_hbm, zeros_hbm, agg_out,
              srcA, dstA, srcB, dstB, rows0, rows1, agg_sh,
              gsem0, gsem1):
    cid = lax.axis_index("c")
    sid = lax.axis_index("s")
    wid = sid * _NC + cid
    r0 = sid * _RPT

    off = 0
    for s in _STAGE:
        pltpu.sync_copy(zeros_hbm.at[pl.ds(r0 + off, s)], rows0.at[pl.ds(0, s)])
        pltpu.sync_copy(rows0.at[pl.ds(0, s)], agg_sh.at[pl.ds(r0 + off, s)])
        off += s
    plsc.subcore_barrier()

    rows = (rows0, rows1)
    sems = (gsem0, gsem1)

    def _load_idx(g, src_g, dst_g):
        pltpu.sync_copy(src_hbm.at[pl.ds(g * _G, _G)], src_g)
        pltpu.sync_copy(dst_hbm.at[pl.ds(g * _G, _G)], dst_g)

    _load_idx(wid, srcA, dstA)
    pltpu.async_copy(h_hbm.at[srcA.at[0]], rows[0], gsem0)

    def _pair(R, carry):
        gA = wid + (2 * R) * _NW
        gB = gA + _NW
        gA_next = gA + 2 * _NW

        @pl.when(gB < _GROUPS)
        def _():
            _load_idx(gB, srcB, dstB)

        for t in range(2 * _G):
            if t == _G:
                @pl.when(gA_next < _GROUPS)
                def _():
                    _load_idx(gA_next, srcA, dstA)

            tn = t + 1
            if tn < _G:
                nsrc, npred_row = srcA, tn
            elif tn < 2 * _G:
                nsrc, npred_row = srcB, tn - _G
            else:
                nsrc, npred_row = srcA, 0
            npred = (gA < _GROUPS) if tn < _G else (
                (gB < _GROUPS) if tn < 2 * _G else (gA_next < _GROUPS))

            @pl.when(npred)
            def _():
                pltpu.async_copy(h_hbm.at[nsrc.at[npred_row]],
                                 rows[tn % 2], sems[tn % 2])

            cpred = (gA < _GROUPS) if t < _G else (gB < _GROUPS)
            cdst = dstA.at[t] if t < _G else dstB.at[t - _G]
            csrc = srcA.at[t] if t < _G else srcB.at[t - _G]

            @pl.when(cpred)
            def _():
                pltpu.make_async_copy(h_hbm.at[csrc], rows[t % 2],
                                      sems[t % 2]).wait()
                pltpu.sync_copy(rows[t % 2], agg_sh.at[cdst], add=True)
        return carry

    lax.fori_loop(0, _PAIRS, _pair, 0)
    plsc.subcore_barrier()

    o0 = cid * _NPAD + r0
    off = 0
    for s in _STAGE:
        pltpu.sync_copy(agg_sh.at[pl.ds(r0 + off, s)], rows0.at[pl.ds(0, s)])
        pltpu.sync_copy(rows0.at[pl.ds(0, s)], agg_out.at[pl.ds(o0 + off, s)])
        off += s


_sc_agg = pl.kernel(
    _agg_body,
    out_type=jax.ShapeDtypeStruct((_NC * _NPAD, _D), jnp.float32),
    mesh=_MESH,
    scratch_types=[
        pltpu.VMEM((_G, _KA), jnp.int32),
        pltpu.VMEM((_G, _KA), jnp.int32),
        pltpu.VMEM((_G, _KA), jnp.int32),
        pltpu.VMEM((_G, _KA), jnp.int32),
        pltpu.VMEM((_KA, _D), jnp.float32),
        pltpu.VMEM((_KA, _D), jnp.float32),
        pltpu.VMEM_SHARED((_NPAD, _D), jnp.float32),
        pltpu.SemaphoreType.DMA,
        pltpu.SemaphoreType.DMA,
    ],
)


def _deg_body(dst_hbm, zeros_hbm, ones_hbm, deg_out,
              dstG, ones_v, deg_sh, sem):
    cid = lax.axis_index("c")
    sid = lax.axis_index("s")
    wid = sid * _NC + cid
    r0 = sid * _RPT

    off = 0
    for s in _STAGE:
        pltpu.sync_copy(zeros_hbm.at[pl.ds(r0 + off, s)], ones_v.at[pl.ds(0, s)])
        pltpu.sync_copy(ones_v.at[pl.ds(0, s)], deg_sh.at[pl.ds(r0 + off, s)])
        off += s
    pltpu.sync_copy(ones_hbm, ones_v)
    plsc.subcore_barrier()

    def _step(G, carry):
        g = wid + G * _NW

        @pl.when(g < _GROUPS)
        def _():
            pltpu.sync_copy(dst_hbm.at[pl.ds(g * _G, _G)], dstG)
            for t in range(_G):
                pltpu.sync_copy(ones_v, deg_sh.at[dstG.at[t]], add=True)
        return carry

    lax.fori_loop(0, _GPW, _step, 0)
    plsc.subcore_barrier()

    o0 = cid * _NPAD + r0
    off = 0
    for s in _STAGE:
        pltpu.sync_copy(deg_sh.at[pl.ds(r0 + off, s)], ones_v.at[pl.ds(0, s)])
        pltpu.sync_copy(ones_v.at[pl.ds(0, s)], deg_out.at[pl.ds(o0 + off, s)])
        off += s


_sc_deg = pl.kernel(
    _deg_body,
    out_type=jax.ShapeDtypeStruct((_NC * _NPAD, _D), jnp.float32),
    mesh=_MESH,
    scratch_types=[
        pltpu.VMEM((_G, _KA), jnp.int32),
        pltpu.VMEM((_KA, _D), jnp.float32),
        pltpu.VMEM_SHARED((_NPAD, _D), jnp.float32),
        pltpu.SemaphoreType.DMA,
    ],
)


def _degred_body(deg_ref, inv_ref):
    deg = deg_ref[0, :, 0] + deg_ref[1, :, 0]
    inv = 1.0 / jnp.maximum(deg, 1.0)
    inv_ref[...] = jnp.broadcast_to(inv[:, None], inv_ref.shape)


_degred = pl.pallas_call(
    _degred_body,
    grid=(_NPAD // _BLK,),
    in_specs=[pl.BlockSpec((_NC, _BLK, _D), lambda i: (0, i, 0))],
    out_specs=pl.BlockSpec((_BLK, 16), lambda i: (i, 0)),
    out_shape=jax.ShapeDtypeStruct((_NPAD, 16), jnp.float32),
)


def _mm0_body(x_ref, w_ref, b_ref, h_ref):
    h_ref[...] = jnp.dot(x_ref[...], w_ref[...],
                         preferred_element_type=jnp.float32) + b_ref[...]


_mm0 = pl.pallas_call(
    _mm0_body,
    grid=(_GRID,),
    in_specs=[
        pl.BlockSpec((_BLK, _D), lambda i: (i, 0)),
        pl.BlockSpec((_D, _D), lambda i: (0, 0)),
        pl.BlockSpec((1, _D), lambda i: (0, 0)),
    ],
    out_specs=pl.BlockSpec((_BLK, _D), lambda i: (i, 0)),
    out_shape=jax.ShapeDtypeStruct((_N, _D), jnp.float32),
)


def _layer_body(agg_ref, inv_ref, cur_ref, w_ref, b_ref, curn_ref, h_ref):
    inv = inv_ref[:, 0]
    a = agg_ref[0] + agg_ref[1]
    curn = jnp.maximum(a * inv[:, None], 0.0) + cur_ref[...]
    curn_ref[...] = curn
    h_ref[...] = jnp.dot(curn, w_ref[...],
                         preferred_element_type=jnp.float32) + b_ref[...]


_layer = pl.pallas_call(
    _layer_body,
    grid=(_GRID,),
    in_specs=[
        pl.BlockSpec((_NC, _BLK, _D), lambda i: (0, i, 0)),
        pl.BlockSpec((_BLK, 16), lambda i: (i, 0)),
        pl.BlockSpec((_BLK, _D), lambda i: (i, 0)),
        pl.BlockSpec((_D, _D), lambda i: (0, 0)),
        pl.BlockSpec((1, _D), lambda i: (0, 0)),
    ],
    out_specs=[
        pl.BlockSpec((_BLK, _D), lambda i: (i, 0)),
        pl.BlockSpec((_BLK, _D), lambda i: (i, 0)),
    ],
    out_shape=[
        jax.ShapeDtypeStruct((_N, _D), jnp.float32),
        jax.ShapeDtypeStruct((_N, _D), jnp.float32),
    ],
)


def _final_body(agg_ref, inv_ref, cur_ref, out_ref):
    inv = inv_ref[:, 0]
    a = agg_ref[0] + agg_ref[1]
    curn = jnp.maximum(a * inv[:, None], 0.0) + cur_ref[...]
    nrm = jnp.sqrt(jnp.sum(curn * curn, axis=-1, keepdims=True))
    out_ref[...] = curn / jnp.maximum(nrm, 1e-12)


_final = pl.pallas_call(
    _final_body,
    grid=(_GRID,),
    in_specs=[
        pl.BlockSpec((_NC, _BLK, _D), lambda i: (0, i, 0)),
        pl.BlockSpec((_BLK, 16), lambda i: (i, 0)),
        pl.BlockSpec((_BLK, _D), lambda i: (i, 0)),
    ],
    out_specs=pl.BlockSpec((_BLK, _D), lambda i: (i, 0)),
    out_shape=jax.ShapeDtypeStruct((_N, _D), jnp.float32),
)


def kernel(x, edge_index, W0, b0, W1, b1, W2, b2):
    src = edge_index[0]
    dst = edge_index[1]
    zeros = jnp.zeros((_NPAD, _D), jnp.float32)
    ones = jnp.ones((_KA, _D), jnp.float32)

    src2d = src.reshape(_CHUNKS_A, _KA)
    dst2d = dst.reshape(_CHUNKS_A, _KA)

    deg_fat = _sc_deg(dst2d, zeros, ones).reshape(_NC, _NPAD, _D)
    inv = _degred(deg_fat)
    h0 = _mm0(x, W0, b0.reshape(1, _D))
    agg0 = _sc_agg(h0, src2d, dst2d, zeros).reshape(_NC, _NPAD, _D)
    cur1, h1 = _layer(agg0, inv, x, W1, b1.reshape(1, _D))
    agg1 = _sc_agg(h1, src2d, dst2d, zeros).reshape(_NC, _NPAD, _D)
    cur2, h2 = _layer(agg1, inv, cur1, W2, b2.reshape(1, _D))
    agg2 = _sc_agg(h2, src2d, dst2d, zeros).reshape(_NC, _NPAD, _D)
    out = _final(agg2, inv, cur2)
    return (out, edge_index)
